# baseline (speedup 1.0000x reference)
import functools

import jax
import jax.numpy as jnp
from jax import lax
from jax.experimental import pallas as pl
from jax.experimental.pallas import tpu as pltpu
from jax.experimental.pallas import tpu_sc as plsc

N = 10000
E = 320000
NP = 10240
IN_CH = 128
HID = 20
OUT = 10
D1 = 24
D2 = 16

NC = 2
NS = 16
NW = NC * NS
EPT = E // NW
K = 125
STEPS = EPT // K
R = 8
G = STEPS // R

RPS = NP // NS


def _sc_mesh():
    return plsc.VectorSubcoreMesh(
        core_axis_name="c", subcore_axis_name="s", num_cores=NC, num_subcores=NS
    )


_SC_PARAMS = pltpu.CompilerParams(
    use_tc_tiling_on_sc=False, needs_layout_passes=False
)


def _agg_ring(tab, src_v, dst_v, acc, bufs, gsems, ssems):
    def gather(j, b):
        pltpu.async_copy(tab.at[src_v.at[j]], bufs[b], gsems[b])

    def gather_wait(b):
        pltpu.make_async_copy(tab.at[src_v.at[0]], bufs[b], gsems[b]).wait()

    def scatter(j, b):
        pltpu.async_copy(bufs[b], acc.at[dst_v.at[j]], ssems[b], add=True)

    def scatter_wait(b):
        pltpu.make_async_copy(bufs[b], acc.at[dst_v.at[0]], ssems[b]).wait()

    for b in range(R):
        gather(b, b)

    def round_fn(g, carry):
        for b in range(R):
            gather_wait(b)
            scatter(g * R + b, b)
        for b in range(R):
            scatter_wait(b)
            jn = (g + 1) * R + b

            @pl.when(jn < STEPS)
            def _():
                gather(jn, b)

        return carry

    lax.fori_loop(0, G, round_fn, 0)


def _rsqrt16(x):
    i = plsc.bitcast(x, jnp.int32)
    i = 0x5F3759DF - lax.shift_right_arithmetic(i, 1)
    y = plsc.bitcast(i, jnp.float32)
    for _ in range(3):
        y = y * (1.5 - 0.5 * x * y * y)
    return y


FSTEPS = E // NS // K
DD = 8


def _fused_pass(h1, dstf3, src3, dst3, ones8, zeros8, zeros):
    @functools.partial(
        pl.kernel,
        out_type=[
            jax.ShapeDtypeStruct((NC, NP, D1), jnp.float32),
            jax.ShapeDtypeStruct((NC, NP, D1), jnp.float32),
            jax.ShapeDtypeStruct((NC, NS, RPS // 16, 16), jnp.float32),
        ],
        mesh=_sc_mesh(),
        compiler_params=_SC_PARAMS,
        scratch_types=(
            [
                pltpu.VMEM((FSTEPS, K), jnp.int32),
                pltpu.VMEM((K, DD), jnp.float32),
                pltpu.VMEM((RPS, DD), jnp.float32),
                pltpu.VMEM((RPS // 16, 16), jnp.float32),
                pltpu.VMEM((RPS, D1), jnp.float32),
                pltpu.VMEM((STEPS, K), jnp.int32),
                pltpu.VMEM((STEPS, K), jnp.int32),
                pltpu.VMEM_SHARED((NP, D1), jnp.float32),
                pltpu.VMEM_SHARED((NP, DD), jnp.float32),
                pltpu.SemaphoreType.DMA,
            ]
            + [pltpu.VMEM((K, D1), jnp.float32) for _ in range(R)]
            + [pltpu.SemaphoreType.DMA for _ in range(2 * R)]
        ),
    )
    def body(h1_hbm, dstf_hbm, src_hbm, dst_hbm, ones_hbm, zeros8_hbm, zeros_hbm,
             agg_hbm, g1_hbm, dis_hbm,
             dstf_v, ones_v, bounce, dv, h1_v, src_v, dst_v, acc, accd, dsem,
             *rest):
        bufs = rest[:R]
        gsems = rest[R : 2 * R]
        ssems = rest[2 * R : 3 * R]
        c = lax.axis_index("c")
        s = lax.axis_index("s")
        wid = s * NC + c
        hrows = RPS // 16

        pltpu.sync_copy(dstf_hbm.at[s], dstf_v)
        pltpu.sync_copy(src_hbm.at[wid], src_v)
        pltpu.sync_copy(dst_hbm.at[wid], dst_v)
        pltpu.sync_copy(ones_hbm, ones_v)
        pltpu.sync_copy(zeros_hbm.at[pl.ds(s * RPS, RPS)], acc.at[pl.ds(s * RPS, RPS)])
        pltpu.sync_copy(zeros8_hbm.at[pl.ds(s * RPS, RPS)], accd.at[pl.ds(s * RPS, RPS)])
        plsc.subcore_barrier()

        LAG = 16

        def dstep(j, carry):
            pltpu.async_copy(ones_v, accd.at[dstf_v.at[j]], dsem, add=True)

            @pl.when(j >= LAG)
            def _():
                pltpu.make_async_copy(ones_v, accd.at[dstf_v.at[0]], dsem).wait()

            return carry

        lax.fori_loop(0, FSTEPS, dstep, 0)

        def ddrain(j, carry):
            pltpu.make_async_copy(ones_v, accd.at[dstf_v.at[0]], dsem).wait()
            return carry

        lax.fori_loop(0, LAG, ddrain, 0)
        plsc.subcore_barrier()

        pltpu.sync_copy(accd.at[pl.ds(s * RPS, RPS)], bounce)
        lanes = lax.iota(jnp.int32, 16)
        cols8 = lax.bitwise_and(lanes, 7)

        def nstep(r, carry):
            deg16 = plsc.load_gather(bounce, [r * 16 + lanes, cols8])
            dv[r, :] = _rsqrt16(deg16 + 1.0)
            return carry

        lax.fori_loop(0, hrows, nstep, 0)
        pltpu.sync_copy(dv, dis_hbm.at[c, s])

        pltpu.sync_copy(h1_hbm.at[pl.ds(s * RPS, RPS)], h1_v)
        lanes = lax.iota(jnp.int32, 16)

        def scale_row(r, carry):
            dvec = dv[r, :]
            rows = r * 16 + lanes
            for f in range(D1):
                fidx = jnp.full((16,), f, jnp.int32)
                col = plsc.load_gather(h1_v, [rows, fidx])
                plsc.store_scatter(h1_v, [rows, fidx], col * dvec)
            return carry

        lax.fori_loop(0, hrows, scale_row, 0)
        pltpu.sync_copy(h1_v, g1_hbm.at[c, pl.ds(s * RPS, RPS)])
        plsc.subcore_barrier()

        _agg_ring(g1_hbm.at[c], src_v, dst_v, acc, bufs, gsems, ssems)
        plsc.subcore_barrier()
        pltpu.sync_copy(acc.at[pl.ds(s * RPS, RPS)], agg_hbm.at[c, pl.ds(s * RPS, RPS)])

    return body(h1, dstf3, src3, dst3, ones8, zeros8, zeros)


def _agg_pass(table, src3, dst3, zeros, d):
    @functools.partial(
        pl.kernel,
        out_type=jax.ShapeDtypeStruct((NC, NP, d), jnp.float32),
        mesh=_sc_mesh(),
        compiler_params=_SC_PARAMS,
        scratch_types=(
            [
                pltpu.VMEM((STEPS, K), jnp.int32),
                pltpu.VMEM((STEPS, K), jnp.int32),
                pltpu.VMEM_SHARED((NP, d), jnp.float32),
            ]
            + [pltpu.VMEM((K, d), jnp.float32) for _ in range(R)]
            + [pltpu.SemaphoreType.DMA for _ in range(2 * R)]
        ),
    )
    def body(tab_hbm, src_hbm, dst_hbm, zeros_hbm, out_hbm, src_v, dst_v, acc, *rest):
        bufs = rest[:R]
        gsems = rest[R : 2 * R]
        ssems = rest[2 * R : 3 * R]
        c = lax.axis_index("c")
        s = lax.axis_index("s")
        wid = s * NC + c
        pltpu.sync_copy(zeros_hbm.at[pl.ds(s * RPS, RPS)], acc.at[pl.ds(s * RPS, RPS)])
        pltpu.sync_copy(src_hbm.at[wid], src_v)
        pltpu.sync_copy(dst_hbm.at[wid], dst_v)
        plsc.subcore_barrier()
        _agg_ring(tab_hbm, src_v, dst_v, acc, bufs, gsems, ssems)
        plsc.subcore_barrier()
        pltpu.sync_copy(acc.at[pl.ds(s * RPS, RPS)], out_hbm.at[c, pl.ds(s * RPS, RPS)])

    return body(table, src3, dst3, zeros)


def _tc_mm1(x, w1p):
    def body(x_ref, w_ref, h_ref):
        h = jnp.dot(x_ref[...], w_ref[...], preferred_element_type=jnp.float32)
        h_ref[...] = jnp.pad(h, ((0, NP - N), (0, 0)))

    return pl.pallas_call(
        body, out_shape=jax.ShapeDtypeStruct((NP, D1), jnp.float32)
    )(x, w1p)


def _tc2(agg1, g1, dis, w2p, b1p):
    def body(agg_ref, g1_ref, dis_ref, w_ref, b_ref, g2_ref):
        a = agg_ref[0] + agg_ref[1] + g1_ref[...]
        h = jnp.maximum(dis_ref[...] * a + b_ref[...], 0.0)
        g2_ref[...] = (
            jnp.dot(h, w_ref[...], preferred_element_type=jnp.float32) * dis_ref[...]
        )

    return pl.pallas_call(
        body, out_shape=jax.ShapeDtypeStruct((NP, D2), jnp.float32)
    )(agg1, g1, dis, w2p, b1p)


def _tc3(agg2, g2, dis, b2p):
    def body(agg_ref, g2_ref, dis_ref, b_ref, out_ref):
        a = agg_ref[0] + agg_ref[1] + g2_ref[...]
        out_ref[...] = lax.slice(dis_ref[...] * a + b_ref[...], (0, 0), (N, OUT))

    return pl.pallas_call(
        body, out_shape=jax.ShapeDtypeStruct((N, OUT), jnp.float32)
    )(agg2, g2, dis, b2p)


def kernel(x, edge_index, W1, b1, W2, b2):
    src3 = edge_index[0].reshape(NW, STEPS, K)
    dst3 = edge_index[1].reshape(NW, STEPS, K)
    dstf3 = edge_index[1].reshape(NS, FSTEPS, K)

    w1p = jnp.pad(W1, ((0, 0), (0, D1 - HID)))
    h1 = _tc_mm1(x, w1p)

    agg1, g1cp, dis4 = _fused_pass(
        h1, dstf3, src3, dst3,
        jnp.ones((K, DD), jnp.float32),
        jnp.zeros((NP, DD), jnp.float32),
        jnp.zeros((NP, D1), jnp.float32),
    )
    g1 = g1cp[0]
    dis = dis4[0].reshape(NP, 1)

    w2p = jnp.pad(W2, ((0, D1 - HID), (0, D2 - OUT)))
    b1p = jnp.pad(b1, (0, D1 - HID)).reshape(1, D1)
    g2 = _tc2(agg1, g1, dis, w2p, b1p)

    agg2 = _agg_pass(g2, src3, dst3, jnp.zeros((NP, D2), jnp.float32), D2)

    b2p = jnp.pad(b2, (0, D2 - OUT)).reshape(1, D2)
    return _tc3(agg2, g2, dis, b2p)


# speedup vs baseline: 1.1338x; 1.1338x over previous
import functools

import jax
import jax.numpy as jnp
from jax import lax
from jax.experimental import pallas as pl
from jax.experimental.pallas import tpu as pltpu
from jax.experimental.pallas import tpu_sc as plsc

N = 10000
E = 320000
NP = 10240
IN_CH = 128
HID = 20
OUT = 10
D1 = 24
D2 = 16
DD = 8

NC = 2
NS = 16
NW = NC * NS
EPT = E // NW
K = 125
STEPS = EPT // K
R = 8
G = STEPS // R


def _sc_mesh():
    return plsc.VectorSubcoreMesh(
        core_axis_name="c", subcore_axis_name="s", num_cores=NC, num_subcores=NS
    )


_SC_PARAMS = pltpu.CompilerParams(
    use_tc_tiling_on_sc=False, needs_layout_passes=False
)


def _deg_pass(dst3, ones, zeros):
    @functools.partial(
        pl.kernel,
        out_type=jax.ShapeDtypeStruct((NC, NP, DD), jnp.float32),
        mesh=_sc_mesh(),
        compiler_params=_SC_PARAMS,
        scratch_types=[
            pltpu.VMEM((STEPS, K), jnp.int32),
            pltpu.VMEM((K, DD), jnp.float32),
            pltpu.VMEM_SHARED((NP, DD), jnp.float32),
            pltpu.SemaphoreType.DMA,
        ],
    )
    def body(dst_hbm, ones_hbm, zeros_hbm, out_hbm, dst_v, ones_v, acc, sem):
        c = lax.axis_index("c")
        s = lax.axis_index("s")
        wid = s * NC + c
        rps = NP // NS
        pltpu.sync_copy(zeros_hbm.at[pl.ds(s * rps, rps)], acc.at[pl.ds(s * rps, rps)])
        pltpu.sync_copy(dst_hbm.at[wid], dst_v)
        pltpu.sync_copy(ones_hbm, ones_v)
        plsc.subcore_barrier()

        LAG = 16

        def step(j, carry):
            pltpu.async_copy(ones_v, acc.at[dst_v.at[j]], sem, add=True)

            @pl.when(j >= LAG)
            def _():
                pltpu.make_async_copy(ones_v, acc.at[dst_v.at[0]], sem).wait()

            return carry

        lax.fori_loop(0, STEPS, step, 0)

        def drain(j, carry):
            pltpu.make_async_copy(ones_v, acc.at[dst_v.at[0]], sem).wait()
            return carry

        lax.fori_loop(0, LAG, drain, 0)
        plsc.subcore_barrier()
        pltpu.sync_copy(acc.at[pl.ds(s * rps, rps)], out_hbm.at[c, pl.ds(s * rps, rps)])

    return body(dst3, ones, zeros)


def _agg_pass(table, src3, dst3, zeros, d):
    @functools.partial(
        pl.kernel,
        out_type=jax.ShapeDtypeStruct((NC, NP, d), jnp.float32),
        mesh=_sc_mesh(),
        compiler_params=_SC_PARAMS,
        scratch_types=(
            [
                pltpu.VMEM((STEPS, K), jnp.int32),
                pltpu.VMEM((STEPS, K), jnp.int32),
                pltpu.VMEM_SHARED((NP, d), jnp.float32),
            ]
            + [pltpu.VMEM((K, d), jnp.float32) for _ in range(R)]
            + [pltpu.SemaphoreType.DMA for _ in range(2 * R)]
        ),
    )
    def body(tab_hbm, src_hbm, dst_hbm, zeros_hbm, out_hbm, src_v, dst_v, acc, *rest):
        bufs = rest[:R]
        gsems = rest[R : 2 * R]
        ssems = rest[2 * R : 3 * R]
        c = lax.axis_index("c")
        s = lax.axis_index("s")
        wid = s * NC + c
        rps = NP // NS
        pltpu.sync_copy(zeros_hbm.at[pl.ds(s * rps, rps)], acc.at[pl.ds(s * rps, rps)])
        pltpu.sync_copy(src_hbm.at[wid], src_v)
        pltpu.sync_copy(dst_hbm.at[wid], dst_v)
        plsc.subcore_barrier()

        def gather(j, b):
            pltpu.async_copy(tab_hbm.at[src_v.at[j]], bufs[b], gsems[b])

        def gather_wait(b):
            pltpu.make_async_copy(tab_hbm.at[src_v.at[0]], bufs[b], gsems[b]).wait()

        def scatter(j, b):
            pltpu.async_copy(bufs[b], acc.at[dst_v.at[j]], ssems[b], add=True)

        def scatter_wait(b):
            pltpu.make_async_copy(bufs[b], acc.at[dst_v.at[0]], ssems[b]).wait()

        for b in range(R):
            gather(b, b)

        def round_fn(g, carry):
            for b in range(R):
                gather_wait(b)
                scatter(g * R + b, b)
            for b in range(R):
                scatter_wait(b)
                jn = (g + 1) * R + b

                @pl.when(jn < STEPS)
                def _():
                    gather(jn, b)

            return carry

        lax.fori_loop(0, G, round_fn, 0)
        plsc.subcore_barrier()
        pltpu.sync_copy(acc.at[pl.ds(s * rps, rps)], out_hbm.at[c, pl.ds(s * rps, rps)])

    return body(table, src3, dst3, zeros)


def _tc1(x, w1p, degp):
    def body(x_ref, w_ref, degp_ref, g_ref, dis_ref):
        deg = degp_ref[0] + degp_ref[1]
        dis = lax.rsqrt(deg[:, 0:1] + 1.0)
        h = jnp.dot(x_ref[...], w_ref[...], preferred_element_type=jnp.float32)
        g_ref[...] = jnp.pad(h, ((0, NP - N), (0, 0))) * dis
        dis_ref[...] = dis

    return pl.pallas_call(
        body,
        out_shape=[
            jax.ShapeDtypeStruct((NP, D1), jnp.float32),
            jax.ShapeDtypeStruct((NP, 1), jnp.float32),
        ],
    )(x, w1p, degp)


def _tc2(agg1, g1, dis, w2p, b1p):
    def body(agg_ref, g1_ref, dis_ref, w_ref, b_ref, g2_ref):
        a = agg_ref[0] + agg_ref[1] + g1_ref[...]
        h = jnp.maximum(dis_ref[...] * a + b_ref[...], 0.0)
        g2_ref[...] = (
            jnp.dot(h, w_ref[...], preferred_element_type=jnp.float32) * dis_ref[...]
        )

    return pl.pallas_call(
        body, out_shape=jax.ShapeDtypeStruct((NP, D2), jnp.float32)
    )(agg1, g1, dis, w2p, b1p)


def _tc3(agg2, g2, dis, b2p):
    def body(agg_ref, g2_ref, dis_ref, b_ref, out_ref):
        a = agg_ref[0] + agg_ref[1] + g2_ref[...]
        out_ref[...] = lax.slice(dis_ref[...] * a + b_ref[...], (0, 0), (N, OUT))

    return pl.pallas_call(
        body, out_shape=jax.ShapeDtypeStruct((N, OUT), jnp.float32)
    )(agg2, g2, dis, b2p)


def kernel(x, edge_index, W1, b1, W2, b2):
    src3 = edge_index[0].reshape(NW, STEPS, K)
    dst3 = edge_index[1].reshape(NW, STEPS, K)

    ones = jnp.ones((K, DD), jnp.float32)
    degp = _deg_pass(dst3, ones, jnp.zeros((NP, DD), jnp.float32))

    w1p = jnp.pad(W1, ((0, 0), (0, D1 - HID)))
    g1, dis = _tc1(x, w1p, degp)

    agg1 = _agg_pass(g1, src3, dst3, jnp.zeros((NP, D1), jnp.float32), D1)

    w2p = jnp.pad(W2, ((0, D1 - HID), (0, D2 - OUT)))
    b1p = jnp.pad(b1, (0, D1 - HID)).reshape(1, D1)
    g2 = _tc2(agg1, g1, dis, w2p, b1p)

    agg2 = _agg_pass(g2, src3, dst3, jnp.zeros((NP, D2), jnp.float32), D2)

    b2p = jnp.pad(b2, (0, D2 - OUT)).reshape(1, D2)
    return _tc3(agg2, g2, dis, b2p)
